# baseline (speedup 1.0000x reference)
import functools

import jax
import jax.numpy as jnp
from jax import lax
from jax.experimental import pallas as pl
from jax.experimental.pallas import tpu as pltpu
from jax.experimental.pallas import tpu_sc as plsc

NC, NS = 2, 16
NW = NC * NS
L = 16

T, N, D, OUT = 3, 100000, 4, 2
E = 3200000
C8 = 2 * T + 2

ER = E // 128
ER_BASE = ER // NW
ER_REM = ER % NW
MB = 16
NMAC = ER_BASE // MB

NODE_STRIDE = 3120
NODE_CHUNK = N - (NW - 1) * NODE_STRIDE
NGROUPS = NODE_CHUNK // L
ACC_ROWS = N // NS

_mesh = plsc.VectorSubcoreMesh(
    core_axis_name="c", subcore_axis_name="s", num_cores=NC, num_subcores=NS
)
_params = pltpu.CompilerParams(
    needs_layout_passes=False, use_tc_tiling_on_sc=False
)


def _wid():
    return lax.axis_index("c") * NS + lax.axis_index("s")


_STG_BR = 200


def _stage_body(e_ref, o_ref):
    for j in range(2):
        for i in range(_STG_BR):
            o_ref[j, i, :] = e_ref[j, pl.ds(i * 128, 128)]


_stage = pl.pallas_call(
    _stage_body,
    out_shape=jax.ShapeDtypeStruct((2, ER, 128), jnp.int32),
    grid=(ER // _STG_BR,),
    in_specs=[pl.BlockSpec((2, _STG_BR * 128), lambda g: (0, g))],
    out_specs=pl.BlockSpec((2, _STG_BR, 128), lambda g: (0, g, 0)),
)


def _proj_body(x_hbm, wflat_hbm, z_hbm, d_hbm, xbuf, zbuf, dbuf, wbuf):
    wid = _wid()
    n0 = wid * NODE_STRIDE
    pltpu.sync_copy(wflat_hbm, wbuf)
    wv = [wbuf[pl.ds(k * L, L)] for k in range(4)]

    def _sc(i):
        return wv[i // L][i % L]

    iota = lax.iota(jnp.int32, L)
    for t in range(T):
        pltpu.sync_copy(x_hbm.at[t, pl.ds(n0, NODE_CHUNK), :], xbuf)
        w1s = [[_sc(t * 8 + dd * 2 + o) for o in range(OUT)] for dd in range(D)]
        w2s = [[_sc(24 + t * 8 + dd * 2 + o) for o in range(OUT)] for dd in range(D)]
        bs = [_sc(48 + t * 2 + o) + _sc(54 + t * 2 + o) for o in range(OUT)]

        def body(g, carry):
            rows = g * L + iota
            xs = [plsc.load_gather(xbuf, [rows, jnp.full((L,), dd, jnp.int32)])
                  for dd in range(D)]
            for o in range(OUT):
                zv = xs[0] * w2s[0][o]
                dv = xs[0] * w1s[0][o]
                for dd in range(1, D):
                    zv = zv + xs[dd] * w2s[dd][o]
                    dv = dv + xs[dd] * w1s[dd][o]
                dv = dv + bs[o]
                ch = jnp.full((L,), 2 * t + o, jnp.int32)
                plsc.store_scatter(zbuf, [rows, ch], zv)
                plsc.store_scatter(dbuf, [rows, ch], dv)
            if t == 0:
                zz = jnp.zeros((L,), jnp.float32)
                for ch in (2 * T, 2 * T + 1):
                    chv = jnp.full((L,), ch, jnp.int32)
                    plsc.store_scatter(zbuf, [rows, chv], zz)
                    plsc.store_scatter(dbuf, [rows, chv], zz)
            return carry

        lax.fori_loop(0, NGROUPS, body, 0)
    pltpu.sync_copy(zbuf, z_hbm.at[pl.ds(n0, NODE_CHUNK), :])
    pltpu.sync_copy(dbuf, d_hbm.at[pl.ds(n0, NODE_CHUNK), :])


_proj = functools.partial(
    pl.kernel,
    out_type=(
        jax.ShapeDtypeStruct((N, C8), jnp.float32),
        jax.ShapeDtypeStruct((N, C8), jnp.float32),
    ),
    mesh=_mesh,
    compiler_params=_params,
    scratch_types=[
        pltpu.VMEM((NODE_CHUNK, D), jnp.float32),
        pltpu.VMEM((NODE_CHUNK, C8), jnp.float32),
        pltpu.VMEM((NODE_CHUNK, C8), jnp.float32),
        pltpu.VMEM((4 * L,), jnp.float32),
    ],
)(_proj_body)


def _scat_body(z_hbm, er_hbm, zero_hbm, parts_hbm,
               sidx, didx, rows, acc, gsem0, gsem1, ssem):
    c = lax.axis_index("c")
    s = lax.axis_index("s")
    wid = c * NS + s
    pltpu.sync_copy(zero_hbm, acc.at[pl.ds(s * ACC_ROWS, ACC_ROWS), :])
    plsc.subcore_barrier()

    r0 = wid * ER_BASE + jnp.minimum(wid, ER_REM)
    cnt = ER_BASE + jnp.where(wid < ER_REM, 1, 0)
    gsems = (gsem0, gsem1)

    def _load_idx(p, r):
        pltpu.sync_copy(er_hbm.at[0, pl.ds(r, MB), :], sidx.at[p])
        pltpu.sync_copy(er_hbm.at[1, pl.ds(r, MB), :], didx.at[p])

    def _si(p, j):
        return sidx.at[p, j]

    def _di(p, j):
        return didx.at[p, j]

    def _fire_gathers(p):
        for j in range(MB):
            pltpu.async_copy(z_hbm.at[_si(p, j)], rows.at[p, j], gsems[p])

    def _wait_gathers(p):
        for j in range(MB):
            pltpu.make_async_copy(
                z_hbm.at[_si(p, j)], rows.at[p, j], gsems[p]
            ).wait()

    def _scatter(p):
        cps = [
            pltpu.async_copy(rows.at[p, j], acc.at[_di(p, j)], ssem, add=True)
            for j in range(MB)
        ]
        for cp in cps:
            cp.wait()

    _load_idx(0, r0)
    _fire_gathers(0)
    _load_idx(1, r0 + MB)
    _fire_gathers(1)

    def mbody(m, carry):
        for b in range(2):
            k = 2 * m + b
            r = r0 + k * MB
            _wait_gathers(b)
            _scatter(b)
            nxt = r + 2 * MB

            @pl.when(k + 2 < NMAC)
            def _():
                _load_idx(b, nxt)
                _fire_gathers(b)

        return carry

    lax.fori_loop(0, NMAC // 2, mbody, 0)

    def tbody(r, carry):
        pltpu.sync_copy(er_hbm.at[0, pl.ds(r, 1), :], sidx.at[0, pl.ds(0, 1), :])
        pltpu.sync_copy(er_hbm.at[1, pl.ds(r, 1), :], didx.at[0, pl.ds(0, 1), :])
        pltpu.async_copy(z_hbm.at[_si(0, 0)], rows.at[0, 0], gsem0).wait()
        pltpu.sync_copy(rows.at[0, 0], acc.at[_di(0, 0)], add=True)
        return carry

    lax.fori_loop(r0 + NMAC * MB, r0 + cnt, tbody, 0)
    plsc.subcore_barrier()
    pltpu.sync_copy(acc.at[pl.ds(s * ACC_ROWS, ACC_ROWS), :],
                    parts_hbm.at[c, pl.ds(s * ACC_ROWS, ACC_ROWS), :])


_scat = functools.partial(
    pl.kernel,
    out_type=jax.ShapeDtypeStruct((NC, N, C8), jnp.float32),
    mesh=_mesh,
    compiler_params=_params,
    scratch_types=[
        pltpu.VMEM((2, MB, 128), jnp.int32),
        pltpu.VMEM((2, MB, 128), jnp.int32),
        pltpu.VMEM((2, MB, 128, C8), jnp.float32),
        pltpu.VMEM_SHARED((N, C8), jnp.float32),
        pltpu.SemaphoreType.DMA,
        pltpu.SemaphoreType.DMA,
        pltpu.SemaphoreType.DMA,
    ],
)(_scat_body)


def _epi_body(d_hbm, parts_hbm, out_hbm, dbuf, p0, p1, ob0, ob1, ob2):
    wid = _wid()
    n0 = wid * NODE_STRIDE
    pltpu.sync_copy(d_hbm.at[pl.ds(n0, NODE_CHUNK), :], dbuf)
    pltpu.sync_copy(parts_hbm.at[0, pl.ds(n0, NODE_CHUNK), :], p0)
    pltpu.sync_copy(parts_hbm.at[1, pl.ds(n0, NODE_CHUNK), :], p1)

    iota = lax.iota(jnp.int32, L)

    obufs = (ob0, ob1, ob2)

    def gbody(g, carry):
        rows = g * L + iota
        for t in range(T):
            for o in range(OUT):
                ch = jnp.full((L,), 2 * t + o, jnp.int32)
                h = (plsc.load_gather(dbuf, [rows, ch])
                     + plsc.load_gather(p0, [rows, ch])
                     + plsc.load_gather(p1, [rows, ch]))
                h = jnp.maximum(h, 0.0)
                sg = 1.0 / (1.0 + jnp.exp(-h))
                plsc.store_scatter(obufs[t], [rows * OUT + o], sg)
        return carry

    lax.fori_loop(0, NGROUPS, gbody, 0)
    for t in range(T):
        pltpu.sync_copy(obufs[t],
                        out_hbm.at[pl.ds(t * N * OUT + n0 * OUT,
                                         NODE_CHUNK * OUT)])


_epi = functools.partial(
    pl.kernel,
    out_type=jax.ShapeDtypeStruct((T * N * OUT,), jnp.float32),
    mesh=_mesh,
    compiler_params=_params,
    scratch_types=[
        pltpu.VMEM((NODE_CHUNK, C8), jnp.float32),
        pltpu.VMEM((NODE_CHUNK, C8), jnp.float32),
        pltpu.VMEM((NODE_CHUNK, C8), jnp.float32),
        pltpu.VMEM((NODE_CHUNK * OUT,), jnp.float32),
        pltpu.VMEM((NODE_CHUNK * OUT,), jnp.float32),
        pltpu.VMEM((NODE_CHUNK * OUT,), jnp.float32),
    ],
)(_epi_body)


def kernel(x, edges, W1, b1, W2, b2):
    zeros = jnp.zeros((ACC_ROWS, C8), jnp.float32)
    wflat = jnp.concatenate([
        W1.reshape(-1), W2.reshape(-1), b1.reshape(-1), b2.reshape(-1),
        jnp.zeros((4,), jnp.float32),
    ])
    er = _stage(edges)
    z, d = _proj(x, wflat)
    parts = _scat(z, er, zeros)
    return _epi(d, parts).reshape(T, N, OUT)


# speedup vs baseline: 1.1186x; 1.1186x over previous
import functools

import jax
import jax.numpy as jnp
from jax import lax
from jax.experimental import pallas as pl
from jax.experimental.pallas import tpu as pltpu
from jax.experimental.pallas import tpu_sc as plsc

NC, NS = 2, 16
NW = NC * NS
L = 16

T, N, D, OUT = 3, 100000, 4, 2
E = 3200000
C8 = 2 * T + 2

ER = E // 128
ER_BASE = ER // NW
ER_REM = ER % NW
MB = 16
NMAC = ER_BASE // MB

ACC_ROWS = N // NS

_mesh = plsc.VectorSubcoreMesh(
    core_axis_name="c", subcore_axis_name="s", num_cores=NC, num_subcores=NS
)
_params = pltpu.CompilerParams(
    needs_layout_passes=False, use_tc_tiling_on_sc=False
)


_PRJ_BN = 2000


def _proj_body(x_ref, w1_ref, b1_ref, w2_ref, b2_ref, z_ref, d_ref):
    zcols = []
    dcols = []
    for t in range(T):
        xt = x_ref[t]
        zcols.append(jnp.dot(xt, w2_ref[t]))
        dcols.append(jnp.dot(xt, w1_ref[t]) + b1_ref[t] + b2_ref[t])
    pad = jnp.zeros((_PRJ_BN, C8 - T * OUT), jnp.float32)
    z_ref[...] = jnp.concatenate(zcols + [pad], axis=1)
    d_ref[...] = jnp.concatenate(dcols + [pad], axis=1)


_proj = pl.pallas_call(
    _proj_body,
    out_shape=(
        jax.ShapeDtypeStruct((N, C8), jnp.float32),
        jax.ShapeDtypeStruct((N, C8), jnp.float32),
    ),
    grid=(N // _PRJ_BN,),
    in_specs=[
        pl.BlockSpec((T, _PRJ_BN, D), lambda g: (0, g, 0)),
        pl.BlockSpec((T, D, OUT), lambda g: (0, 0, 0)),
        pl.BlockSpec((T, OUT), lambda g: (0, 0)),
        pl.BlockSpec((T, D, OUT), lambda g: (0, 0, 0)),
        pl.BlockSpec((T, OUT), lambda g: (0, 0)),
    ],
    out_specs=(
        pl.BlockSpec((_PRJ_BN, C8), lambda g: (g, 0)),
        pl.BlockSpec((_PRJ_BN, C8), lambda g: (g, 0)),
    ),
)


_STG_BR = 200


def _stage_body(e_ref, o_ref):
    for j in range(2):
        for i in range(_STG_BR):
            o_ref[j, i, :] = e_ref[j, pl.ds(i * 128, 128)]


_stage = pl.pallas_call(
    _stage_body,
    out_shape=jax.ShapeDtypeStruct((2, ER, 128), jnp.int32),
    grid=(ER // _STG_BR,),
    in_specs=[pl.BlockSpec((2, _STG_BR * 128), lambda g: (0, g))],
    out_specs=pl.BlockSpec((2, _STG_BR, 128), lambda g: (0, g, 0)),
)


def _scat_body(z_hbm, er_hbm, zero_hbm, parts_hbm,
               sidx, didx, rows, acc, gsem0, gsem1, ssem):
    c = lax.axis_index("c")
    s = lax.axis_index("s")
    wid = c * NS + s
    pltpu.sync_copy(zero_hbm, acc.at[pl.ds(s * ACC_ROWS, ACC_ROWS), :])
    plsc.subcore_barrier()

    r0 = wid * ER_BASE + jnp.minimum(wid, ER_REM)
    cnt = ER_BASE + jnp.where(wid < ER_REM, 1, 0)
    gsems = (gsem0, gsem1)

    def _load_idx(p, r):
        pltpu.sync_copy(er_hbm.at[0, pl.ds(r, MB), :], sidx.at[p])
        pltpu.sync_copy(er_hbm.at[1, pl.ds(r, MB), :], didx.at[p])

    def _fire_gathers(p):
        for j in range(MB):
            pltpu.async_copy(z_hbm.at[sidx.at[p, j]], rows.at[p, j], gsems[p])

    def _wait_gathers(p):
        for j in range(MB):
            pltpu.make_async_copy(
                z_hbm.at[sidx.at[p, j]], rows.at[p, j], gsems[p]
            ).wait()

    def _scatter(p):
        cps = [
            pltpu.async_copy(rows.at[p, j], acc.at[didx.at[p, j]], ssem,
                             add=True)
            for j in range(MB)
        ]
        for cp in cps:
            cp.wait()

    _load_idx(0, r0)
    _fire_gathers(0)
    _load_idx(1, r0 + MB)
    _fire_gathers(1)

    def mbody(m, carry):
        for b in range(2):
            k = 2 * m + b
            r = r0 + k * MB
            _wait_gathers(b)
            _scatter(b)
            nxt = r + 2 * MB

            @pl.when(k + 2 < NMAC)
            def _():
                _load_idx(b, nxt)
                _fire_gathers(b)

        return carry

    lax.fori_loop(0, NMAC // 2, mbody, 0)

    def tbody(r, carry):
        pltpu.sync_copy(er_hbm.at[0, pl.ds(r, 1), :], sidx.at[0, pl.ds(0, 1), :])
        pltpu.sync_copy(er_hbm.at[1, pl.ds(r, 1), :], didx.at[0, pl.ds(0, 1), :])
        pltpu.async_copy(z_hbm.at[sidx.at[0, 0]], rows.at[0, 0], gsem0).wait()
        pltpu.sync_copy(rows.at[0, 0], acc.at[didx.at[0, 0]], add=True)
        return carry

    lax.fori_loop(r0 + NMAC * MB, r0 + cnt, tbody, 0)
    plsc.subcore_barrier()
    pltpu.sync_copy(acc.at[pl.ds(s * ACC_ROWS, ACC_ROWS), :],
                    parts_hbm.at[c, pl.ds(s * ACC_ROWS, ACC_ROWS), :])


_scat = functools.partial(
    pl.kernel,
    out_type=jax.ShapeDtypeStruct((NC, N, C8), jnp.float32),
    mesh=_mesh,
    compiler_params=_params,
    scratch_types=[
        pltpu.VMEM((2, MB, 128), jnp.int32),
        pltpu.VMEM((2, MB, 128), jnp.int32),
        pltpu.VMEM((2, MB, 128, C8), jnp.float32),
        pltpu.VMEM_SHARED((N, C8), jnp.float32),
        pltpu.SemaphoreType.DMA,
        pltpu.SemaphoreType.DMA,
        pltpu.SemaphoreType.DMA,
    ],
)(_scat_body)


_EPI_BN = 2000


def _epi_body(d_ref, parts_ref, o_ref):
    h = d_ref[...] + parts_ref[0] + parts_ref[1]
    sg = jax.nn.sigmoid(jnp.maximum(h, 0.0))
    for t in range(T):
        o_ref[t] = sg[:, 2 * t:2 * t + OUT]


_epi = pl.pallas_call(
    _epi_body,
    out_shape=jax.ShapeDtypeStruct((T, N, OUT), jnp.float32),
    grid=(N // _EPI_BN,),
    in_specs=[
        pl.BlockSpec((_EPI_BN, C8), lambda g: (g, 0)),
        pl.BlockSpec((NC, _EPI_BN, C8), lambda g: (0, g, 0)),
    ],
    out_specs=pl.BlockSpec((T, _EPI_BN, OUT), lambda g: (0, g, 0)),
)


def kernel(x, edges, W1, b1, W2, b2):
    zeros = jnp.zeros((ACC_ROWS, C8), jnp.float32)
    er = _stage(edges)
    z, d = _proj(x, W1, b1, W2, b2)
    parts = _scat(z, er, zeros)
    return _epi(d, parts)


# speedup vs baseline: 1.1515x; 1.0294x over previous
import functools

import jax
import jax.numpy as jnp
from jax import lax
from jax.experimental import pallas as pl
from jax.experimental.pallas import tpu as pltpu
from jax.experimental.pallas import tpu_sc as plsc

NC, NS = 2, 16
NW = NC * NS
L = 16

T, N, D, OUT = 3, 100000, 4, 2
E = 3200000
C8 = 2 * T + 2

ER = E // 128
ER_BASE = ER // NW
ER_REM = ER % NW
MB = 24
NMAC = ER_BASE // MB

ACC_ROWS = N // NS

_mesh = plsc.VectorSubcoreMesh(
    core_axis_name="c", subcore_axis_name="s", num_cores=NC, num_subcores=NS
)
_params = pltpu.CompilerParams(
    needs_layout_passes=False, use_tc_tiling_on_sc=False
)


_PRJ_BN = 2000


def _proj_body(x_ref, w1p_ref, w2p_ref, bp_ref, z_ref, d_ref):
    zacc = dacc = None
    for t in range(T):
        xt = x_ref[t]
        zt = jnp.dot(xt, w2p_ref[t])
        dt = jnp.dot(xt, w1p_ref[t])
        zacc = zt if zacc is None else zacc + zt
        dacc = dt if dacc is None else dacc + dt
    z_ref[...] = zacc
    d_ref[...] = dacc + bp_ref[...]


_proj = pl.pallas_call(
    _proj_body,
    out_shape=(
        jax.ShapeDtypeStruct((N, C8), jnp.float32),
        jax.ShapeDtypeStruct((N, C8), jnp.float32),
    ),
    grid=(N // _PRJ_BN,),
    in_specs=[
        pl.BlockSpec((T, _PRJ_BN, D), lambda g: (0, g, 0)),
        pl.BlockSpec((T, D, C8), lambda g: (0, 0, 0)),
        pl.BlockSpec((T, D, C8), lambda g: (0, 0, 0)),
        pl.BlockSpec((1, C8), lambda g: (0, 0)),
    ],
    out_specs=(
        pl.BlockSpec((_PRJ_BN, C8), lambda g: (g, 0)),
        pl.BlockSpec((_PRJ_BN, C8), lambda g: (g, 0)),
    ),
)


_STG_BR = 200


def _stage_body(e_ref, o_ref):
    for j in range(2):
        for i in range(_STG_BR):
            o_ref[j, i, :] = e_ref[j, pl.ds(i * 128, 128)]


_stage = pl.pallas_call(
    _stage_body,
    out_shape=jax.ShapeDtypeStruct((2, ER, 128), jnp.int32),
    grid=(ER // _STG_BR,),
    in_specs=[pl.BlockSpec((2, _STG_BR * 128), lambda g: (0, g))],
    out_specs=pl.BlockSpec((2, _STG_BR, 128), lambda g: (0, g, 0)),
)


def _scat_body(z_hbm, er_hbm, zero_hbm, parts_hbm,
               sidx, didx, rows, acc, gsem0, gsem1, ssem):
    c = lax.axis_index("c")
    s = lax.axis_index("s")
    wid = c * NS + s
    pltpu.sync_copy(zero_hbm, acc.at[pl.ds(s * ACC_ROWS, ACC_ROWS), :])
    plsc.subcore_barrier()

    r0 = wid * ER_BASE + jnp.minimum(wid, ER_REM)
    cnt = ER_BASE + jnp.where(wid < ER_REM, 1, 0)
    gsems = (gsem0, gsem1)

    def _load_idx(p, r):
        pltpu.sync_copy(er_hbm.at[0, pl.ds(r, MB), :], sidx.at[p])
        pltpu.sync_copy(er_hbm.at[1, pl.ds(r, MB), :], didx.at[p])

    def _fire_gathers(p):
        for j in range(MB):
            pltpu.async_copy(z_hbm.at[sidx.at[p, j]], rows.at[p, j], gsems[p])

    def _wait_gathers(p):
        for j in range(MB):
            pltpu.make_async_copy(
                z_hbm.at[sidx.at[p, j]], rows.at[p, j], gsems[p]
            ).wait()

    def _scatter(p):
        cps = [
            pltpu.async_copy(rows.at[p, j], acc.at[didx.at[p, j]], ssem,
                             add=True)
            for j in range(MB)
        ]
        for cp in cps:
            cp.wait()

    _load_idx(0, r0)
    _fire_gathers(0)
    _load_idx(1, r0 + MB)
    _fire_gathers(1)

    def mbody(m, carry):
        for b in range(2):
            k = 2 * m + b
            r = r0 + k * MB
            _wait_gathers(b)
            _scatter(b)
            nxt = r + 2 * MB

            @pl.when(k + 2 < NMAC)
            def _():
                _load_idx(b, nxt)
                _fire_gathers(b)

        return carry

    lax.fori_loop(0, NMAC // 2, mbody, 0)

    def tbody(r, carry):
        pltpu.sync_copy(er_hbm.at[0, pl.ds(r, 1), :], sidx.at[0, pl.ds(0, 1), :])
        pltpu.sync_copy(er_hbm.at[1, pl.ds(r, 1), :], didx.at[0, pl.ds(0, 1), :])
        pltpu.async_copy(z_hbm.at[sidx.at[0, 0]], rows.at[0, 0], gsem0).wait()
        pltpu.sync_copy(rows.at[0, 0], acc.at[didx.at[0, 0]], add=True)
        return carry

    lax.fori_loop(r0 + NMAC * MB, r0 + cnt, tbody, 0)
    plsc.subcore_barrier()
    pltpu.sync_copy(acc.at[pl.ds(s * ACC_ROWS, ACC_ROWS), :],
                    parts_hbm.at[c, pl.ds(s * ACC_ROWS, ACC_ROWS), :])


_scat = functools.partial(
    pl.kernel,
    out_type=jax.ShapeDtypeStruct((NC, N, C8), jnp.float32),
    mesh=_mesh,
    compiler_params=_params,
    scratch_types=[
        pltpu.VMEM((2, MB, 128), jnp.int32),
        pltpu.VMEM((2, MB, 128), jnp.int32),
        pltpu.VMEM((2, MB, 128, C8), jnp.float32),
        pltpu.VMEM_SHARED((N, C8), jnp.float32),
        pltpu.SemaphoreType.DMA,
        pltpu.SemaphoreType.DMA,
        pltpu.SemaphoreType.DMA,
    ],
)(_scat_body)


_EPI_BN = 2000


def _epi_body(d_ref, parts_ref, o_ref):
    h = d_ref[...] + parts_ref[0] + parts_ref[1]
    sg = jax.nn.sigmoid(jnp.maximum(h, 0.0))
    for t in range(T):
        o_ref[t] = sg[:, 2 * t:2 * t + OUT]


_epi = pl.pallas_call(
    _epi_body,
    out_shape=jax.ShapeDtypeStruct((T, N, OUT), jnp.float32),
    grid=(N // _EPI_BN,),
    in_specs=[
        pl.BlockSpec((_EPI_BN, C8), lambda g: (g, 0)),
        pl.BlockSpec((NC, _EPI_BN, C8), lambda g: (0, g, 0)),
    ],
    out_specs=pl.BlockSpec((T, _EPI_BN, OUT), lambda g: (0, g, 0)),
)


def kernel(x, edges, W1, b1, W2, b2):
    zeros = jnp.zeros((ACC_ROWS, C8), jnp.float32)
    w1p = jnp.zeros((T, D, C8), jnp.float32)
    w2p = jnp.zeros((T, D, C8), jnp.float32)
    bp = jnp.zeros((T, C8), jnp.float32)
    for t in range(T):
        w1p = w1p.at[t, :, 2 * t:2 * t + OUT].set(W1[t])
        w2p = w2p.at[t, :, 2 * t:2 * t + OUT].set(W2[t])
        bp = bp.at[t, 2 * t:2 * t + OUT].set(b1[t] + b2[t])
    bpr = jnp.sum(bp, axis=0, keepdims=True)
    er = _stage(edges)
    z, d = _proj(x, w1p, w2p, bpr)
    parts = _scat(z, er, zeros)
    return _epi(d, parts)


# speedup vs baseline: 1.2425x; 1.0790x over previous
import functools

import jax
import jax.numpy as jnp
from jax import lax
from jax.experimental import pallas as pl
from jax.experimental.pallas import tpu as pltpu
from jax.experimental.pallas import tpu_sc as plsc

NC, NS = 2, 16
NW = NC * NS
L = 16

T, N, D, OUT = 3, 100000, 4, 2
E = 3200000
C8 = 2 * T + 2

ER = E // 128
ER_BASE = ER // NW
ER_REM = ER % NW
MB = 24
NMAC = ER_BASE // MB

ACC_ROWS = N // NS

_mesh = plsc.VectorSubcoreMesh(
    core_axis_name="c", subcore_axis_name="s", num_cores=NC, num_subcores=NS
)
_params = pltpu.CompilerParams(
    needs_layout_passes=False, use_tc_tiling_on_sc=False
)


_PRJ_BN = 5000


def _proj_body(x_ref, w1p_ref, w2p_ref, bp_ref, z_ref, d_ref):
    zacc = dacc = None
    for t in range(T):
        xt = x_ref[t]
        zt = jnp.dot(xt, w2p_ref[t])
        dt = jnp.dot(xt, w1p_ref[t])
        zacc = zt if zacc is None else zacc + zt
        dacc = dt if dacc is None else dacc + dt
    z_ref[...] = zacc
    d_ref[...] = dacc + bp_ref[...]


_proj = pl.pallas_call(
    _proj_body,
    out_shape=(
        jax.ShapeDtypeStruct((N, C8), jnp.float32),
        jax.ShapeDtypeStruct((N, C8), jnp.float32),
    ),
    grid=(N // _PRJ_BN,),
    in_specs=[
        pl.BlockSpec((T, _PRJ_BN, D), lambda g: (0, g, 0)),
        pl.BlockSpec((T, D, C8), lambda g: (0, 0, 0)),
        pl.BlockSpec((T, D, C8), lambda g: (0, 0, 0)),
        pl.BlockSpec((1, C8), lambda g: (0, 0)),
    ],
    out_specs=(
        pl.BlockSpec((_PRJ_BN, C8), lambda g: (g, 0)),
        pl.BlockSpec((_PRJ_BN, C8), lambda g: (g, 0)),
    ),
)


_STG_BR = 1000


def _stage_body(e_ref, o_ref):
    for j in range(2):
        for i in range(_STG_BR):
            o_ref[j, i, :] = e_ref[j, pl.ds(i * 128, 128)]


_stage = pl.pallas_call(
    _stage_body,
    out_shape=jax.ShapeDtypeStruct((2, ER, 128), jnp.int32),
    grid=(ER // _STG_BR,),
    in_specs=[pl.BlockSpec((2, _STG_BR * 128), lambda g: (0, g))],
    out_specs=pl.BlockSpec((2, _STG_BR, 128), lambda g: (0, g, 0)),
)


def _scat_body(z_hbm, er_hbm, zero_hbm, parts_hbm,
               sidx, didx, rows, acc, gsem0, gsem1, ssem):
    c = lax.axis_index("c")
    s = lax.axis_index("s")
    wid = c * NS + s
    pltpu.sync_copy(zero_hbm, acc.at[pl.ds(s * ACC_ROWS, ACC_ROWS), :])
    plsc.subcore_barrier()

    r0 = wid * ER_BASE + jnp.minimum(wid, ER_REM)
    cnt = ER_BASE + jnp.where(wid < ER_REM, 1, 0)
    gsems = (gsem0, gsem1)

    def _load_idx(p, r):
        pltpu.sync_copy(er_hbm.at[0, pl.ds(r, MB), :], sidx.at[p])
        pltpu.sync_copy(er_hbm.at[1, pl.ds(r, MB), :], didx.at[p])

    def _fire_gathers(p):
        for j in range(MB):
            pltpu.async_copy(z_hbm.at[sidx.at[p, j]], rows.at[p, j], gsems[p])

    def _wait_gathers(p):
        for j in range(MB):
            pltpu.make_async_copy(
                z_hbm.at[sidx.at[p, j]], rows.at[p, j], gsems[p]
            ).wait()

    def _scatter(p):
        cps = [
            pltpu.async_copy(rows.at[p, j], acc.at[didx.at[p, j]], ssem,
                             add=True)
            for j in range(MB)
        ]
        for cp in cps:
            cp.wait()

    _load_idx(0, r0)
    _fire_gathers(0)
    _load_idx(1, r0 + MB)
    _fire_gathers(1)

    def mbody(m, carry):
        for b in range(2):
            k = 2 * m + b
            r = r0 + k * MB
            _wait_gathers(b)
            _scatter(b)
            nxt = r + 2 * MB

            @pl.when(k + 2 < NMAC)
            def _():
                _load_idx(b, nxt)
                _fire_gathers(b)

        return carry

    lax.fori_loop(0, NMAC // 2, mbody, 0)

    def tbody(r, carry):
        pltpu.sync_copy(er_hbm.at[0, pl.ds(r, 1), :], sidx.at[0, pl.ds(0, 1), :])
        pltpu.sync_copy(er_hbm.at[1, pl.ds(r, 1), :], didx.at[0, pl.ds(0, 1), :])
        pltpu.async_copy(z_hbm.at[sidx.at[0, 0]], rows.at[0, 0], gsem0).wait()
        pltpu.sync_copy(rows.at[0, 0], acc.at[didx.at[0, 0]], add=True)
        return carry

    lax.fori_loop(r0 + NMAC * MB, r0 + cnt, tbody, 0)
    plsc.subcore_barrier()
    pltpu.sync_copy(acc.at[pl.ds(s * ACC_ROWS, ACC_ROWS), :],
                    parts_hbm.at[c, pl.ds(s * ACC_ROWS, ACC_ROWS), :])


_scat = functools.partial(
    pl.kernel,
    out_type=jax.ShapeDtypeStruct((NC, N, C8), jnp.float32),
    mesh=_mesh,
    compiler_params=_params,
    scratch_types=[
        pltpu.VMEM((2, MB, 128), jnp.int32),
        pltpu.VMEM((2, MB, 128), jnp.int32),
        pltpu.VMEM((2, MB, 128, C8), jnp.float32),
        pltpu.VMEM_SHARED((N, C8), jnp.float32),
        pltpu.SemaphoreType.DMA,
        pltpu.SemaphoreType.DMA,
        pltpu.SemaphoreType.DMA,
    ],
)(_scat_body)


_EPI_BN = 4000


def _epi_body(d_ref, parts_ref, o_ref):
    h = d_ref[...] + parts_ref[0] + parts_ref[1]
    sg = jax.nn.sigmoid(jnp.maximum(h, 0.0))
    for t in range(T):
        o_ref[t] = sg[:, 2 * t:2 * t + OUT]


_epi = pl.pallas_call(
    _epi_body,
    out_shape=jax.ShapeDtypeStruct((T, N, OUT), jnp.float32),
    grid=(N // _EPI_BN,),
    in_specs=[
        pl.BlockSpec((_EPI_BN, C8), lambda g: (g, 0)),
        pl.BlockSpec((NC, _EPI_BN, C8), lambda g: (0, g, 0)),
    ],
    out_specs=pl.BlockSpec((T, _EPI_BN, OUT), lambda g: (0, g, 0)),
)


def kernel(x, edges, W1, b1, W2, b2):
    zeros = jnp.zeros((ACC_ROWS, C8), jnp.float32)
    w1p = jnp.zeros((T, D, C8), jnp.float32)
    w2p = jnp.zeros((T, D, C8), jnp.float32)
    bp = jnp.zeros((T, C8), jnp.float32)
    for t in range(T):
        w1p = w1p.at[t, :, 2 * t:2 * t + OUT].set(W1[t])
        w2p = w2p.at[t, :, 2 * t:2 * t + OUT].set(W2[t])
        bp = bp.at[t, 2 * t:2 * t + OUT].set(b1[t] + b2[t])
    bpr = jnp.sum(bp, axis=0, keepdims=True)
    er = _stage(edges)
    z, d = _proj(x, w1p, w2p, bpr)
    parts = _scat(z, er, zeros)
    return _epi(d, parts)
